# baseline (speedup 1.0000x reference)
import functools

import jax
import jax.numpy as jnp
from jax.experimental import pallas as pl
from jax.experimental.pallas import tpu as pltpu

_PAD_IDX = 0
_LANE = 128
_SUBLANE = 8
_TB = 512
_CHUNK = 64
_SLOTS = 4
_MAX_TOKENS_PER_CALL = 32768


def _round_up(a, b):
    return (a + b - 1) // b * b


def _gather_kernel(idx_ref, w_hbm, out_ref, sems, *, tb, chunk, slots):
    base = pl.program_id(0) * tb
    n_chunks = tb // chunk

    def issue(c):
        slot = c % slots
        for k in range(chunk):
            r = c * chunk + k
            row = idx_ref[base + r]
            pltpu.make_async_copy(
                w_hbm.at[row],
                out_ref.at[r],
                sems.at[slot],
            ).start(priority=c % 2)

    def wait(c):
        pltpu.make_async_copy(
            w_hbm.at[pl.ds(0, chunk)],
            out_ref.at[pl.ds(c * chunk, chunk)],
            sems.at[c % slots],
        ).wait()

    depth = min(slots - 1, n_chunks)
    for c in range(depth):
        issue(c)
    for c in range(n_chunks):
        if c + depth < n_chunks:
            issue(c + depth)
        wait(c)


def _lookup_hbm_gather(flat_ids, weights3, tb):
    n_tok = flat_ids.shape[0]
    if n_tok > _MAX_TOKENS_PER_CALL:
        parts = [
            _lookup_hbm_gather(flat_ids[s:s + _MAX_TOKENS_PER_CALL], weights3, tb)
            for s in range(0, n_tok, _MAX_TOKENS_PER_CALL)
        ]
        return jnp.concatenate(parts, axis=0)

    vsz, s, _ = weights3.shape
    n_pad = _round_up(n_tok, tb)
    nb = n_pad // tb
    if tb % _CHUNK == 0:
        chunk = _CHUNK
    elif tb % 32 == 0:
        chunk = 32
    else:
        chunk = _SUBLANE

    ids = jnp.pad(flat_ids, (0, n_pad - n_tok), constant_values=_PAD_IDX)

    out = pl.pallas_call(
        functools.partial(_gather_kernel, tb=tb, chunk=chunk, slots=_SLOTS),
        out_shape=jax.ShapeDtypeStruct((n_pad, s, _LANE), weights3.dtype),
        grid_spec=pltpu.PrefetchScalarGridSpec(
            num_scalar_prefetch=1,
            grid=(nb,),
            in_specs=[pl.BlockSpec(memory_space=pl.ANY)],
            out_specs=pl.BlockSpec((tb, s, _LANE), lambda i, idx: (i, 0, 0)),
            scratch_shapes=[pltpu.SemaphoreType.DMA((_SLOTS,))],
        ),
        compiler_params=pltpu.CompilerParams(
            dimension_semantics=("parallel",),
        ),
    )(ids, weights3)
    return out[:n_tok]


def kernel(x, weights):
    B, T = x.shape
    vsz, dsz = weights.shape

    flat_ids = jnp.clip(x.reshape(-1).astype(jnp.int32), 0, vsz - 1)
    n_tok = flat_ids.shape[0]

    if dsz % (8 * _LANE) == 0:
        w3 = weights.reshape(vsz, dsz // _LANE, _LANE)
    else:
        w3 = weights.reshape(vsz, 1, dsz)

    tb = _round_up(min(_TB, _round_up(n_tok, _SUBLANE)), _SUBLANE)
    out_flat = _lookup_hbm_gather(flat_ids, w3, tb)
    return out_flat.reshape(B, T, dsz)


# speedup vs baseline: 2.1074x; 2.1074x over previous
import functools

import jax
import jax.numpy as jnp
from jax.experimental import pallas as pl
from jax.experimental.pallas import tpu as pltpu

_PAD_IDX = 0
_SUBLANE = 8
_TB = 512
_CHUNK = 32
_SLOTS = 8
_MAX_TOKENS_PER_CALL = 32768


def _round_up(a, b):
    return (a + b - 1) // b * b


def _gather_kernel(idx_ref, w_hbm, out_ref, sems, *, tb, chunk, slots):
    base = pl.program_id(0) * tb
    n_chunks = tb // chunk

    def issue(c):
        slot = c % slots
        for k in range(chunk):
            r = c * chunk + k
            row = idx_ref[base + r]
            pltpu.make_async_copy(
                w_hbm.at[pl.ds(row, 1), :],
                out_ref.at[pl.ds(r, 1), :],
                sems.at[slot],
            ).start(priority=c % 2)

    def wait(c):
        pltpu.make_async_copy(
            w_hbm.at[pl.ds(0, chunk), :],
            out_ref.at[pl.ds(c * chunk, chunk), :],
            sems.at[c % slots],
        ).wait()

    depth = min(slots - 1, n_chunks)
    for c in range(depth):
        issue(c)
    for c in range(n_chunks):
        if c + depth < n_chunks:
            issue(c + depth)
        wait(c)


def _lookup_hbm_gather(flat_ids, weights, tb):
    n_tok = flat_ids.shape[0]
    if n_tok > _MAX_TOKENS_PER_CALL:
        parts = [
            _lookup_hbm_gather(flat_ids[s:s + _MAX_TOKENS_PER_CALL], weights, tb)
            for s in range(0, n_tok, _MAX_TOKENS_PER_CALL)
        ]
        return jnp.concatenate(parts, axis=0)

    vsz, dsz = weights.shape
    n_pad = _round_up(n_tok, tb)
    nb = n_pad // tb
    if tb % _CHUNK == 0:
        chunk = _CHUNK
    elif tb % 32 == 0:
        chunk = 32
    else:
        chunk = _SUBLANE

    ids = jnp.pad(flat_ids, (0, n_pad - n_tok), constant_values=_PAD_IDX)

    out = pl.pallas_call(
        functools.partial(_gather_kernel, tb=tb, chunk=chunk, slots=_SLOTS),
        out_shape=jax.ShapeDtypeStruct((n_pad, dsz), weights.dtype),
        grid_spec=pltpu.PrefetchScalarGridSpec(
            num_scalar_prefetch=1,
            grid=(nb,),
            in_specs=[pl.BlockSpec(memory_space=pl.ANY)],
            out_specs=pl.BlockSpec((tb, dsz), lambda i, idx: (i, 0)),
            scratch_shapes=[pltpu.SemaphoreType.DMA((_SLOTS,))],
        ),
        compiler_params=pltpu.CompilerParams(
            dimension_semantics=("parallel",),
        ),
    )(ids, weights)
    return out[:n_tok]


def kernel(x, weights):
    B, T = x.shape
    vsz, dsz = weights.shape

    flat_ids = jnp.clip(x.reshape(-1).astype(jnp.int32), 0, vsz - 1)
    n_tok = flat_ids.shape[0]

    tb = _round_up(min(_TB, _round_up(n_tok, _SUBLANE)), _SUBLANE)
    out_flat = _lookup_hbm_gather(flat_ids, weights, tb)
    return out_flat.reshape(B, T, dsz)


# speedup vs baseline: 2.1398x; 1.0154x over previous
import functools

import jax
import jax.numpy as jnp
from jax.experimental import pallas as pl
from jax.experimental.pallas import tpu as pltpu

_PAD_IDX = 0
_SUBLANE = 8
_TB = 512
_CHUNK = 16
_SLOTS = 16
_MAX_TOKENS_PER_CALL = 32768


def _round_up(a, b):
    return (a + b - 1) // b * b


def _gather_kernel(idx_ref, w_hbm, out_ref, sems, *, tb, chunk, slots):
    base = pl.program_id(0) * tb
    n_chunks = tb // chunk

    def issue(c):
        slot = c % slots
        for k in range(chunk):
            r = c * chunk + k
            row = idx_ref[base + r]
            pltpu.make_async_copy(
                w_hbm.at[pl.ds(row, 1), :],
                out_ref.at[pl.ds(r, 1), :],
                sems.at[slot],
            ).start(priority=c % 2)

    def wait(c):
        pltpu.make_async_copy(
            w_hbm.at[pl.ds(0, chunk), :],
            out_ref.at[pl.ds(c * chunk, chunk), :],
            sems.at[c % slots],
        ).wait()

    depth = min(slots - 1, n_chunks)
    for c in range(depth):
        issue(c)
    for c in range(n_chunks):
        if c + depth < n_chunks:
            issue(c + depth)
        wait(c)


def _lookup_hbm_gather(flat_ids, weights, tb):
    n_tok = flat_ids.shape[0]
    if n_tok > _MAX_TOKENS_PER_CALL:
        parts = [
            _lookup_hbm_gather(flat_ids[s:s + _MAX_TOKENS_PER_CALL], weights, tb)
            for s in range(0, n_tok, _MAX_TOKENS_PER_CALL)
        ]
        return jnp.concatenate(parts, axis=0)

    vsz, dsz = weights.shape
    n_pad = _round_up(n_tok, tb)
    nb = n_pad // tb
    if tb % _CHUNK == 0:
        chunk = _CHUNK
    elif tb % 32 == 0:
        chunk = 32
    else:
        chunk = _SUBLANE

    ids = jnp.pad(flat_ids, (0, n_pad - n_tok), constant_values=_PAD_IDX)

    out = pl.pallas_call(
        functools.partial(_gather_kernel, tb=tb, chunk=chunk, slots=_SLOTS),
        out_shape=jax.ShapeDtypeStruct((n_pad, dsz), weights.dtype),
        grid_spec=pltpu.PrefetchScalarGridSpec(
            num_scalar_prefetch=1,
            grid=(nb,),
            in_specs=[pl.BlockSpec(memory_space=pl.ANY)],
            out_specs=pl.BlockSpec((tb, dsz), lambda i, idx: (i, 0)),
            scratch_shapes=[pltpu.SemaphoreType.DMA((_SLOTS,))],
        ),
        compiler_params=pltpu.CompilerParams(
            dimension_semantics=("parallel",),
        ),
    )(ids, weights)
    return out[:n_tok]


def kernel(x, weights):
    B, T = x.shape
    vsz, dsz = weights.shape

    flat_ids = jnp.clip(x.reshape(-1).astype(jnp.int32), 0, vsz - 1)
    n_tok = flat_ids.shape[0]

    tb = _round_up(min(_TB, _round_up(n_tok, _SUBLANE)), _SUBLANE)
    out_flat = _lookup_hbm_gather(flat_ids, weights, tb)
    return out_flat.reshape(B, T, dsz)


# speedup vs baseline: 2.4909x; 1.1641x over previous
import functools

import jax
import jax.numpy as jnp
from jax.experimental import pallas as pl
from jax.experimental.pallas import tpu as pltpu

_PAD_IDX = 0
_SUBLANE = 8
_TB = 1024
_CHUNK = 16
_SLOTS = 16
_MAX_TOKENS_PER_CALL = 32768


def _round_up(a, b):
    return (a + b - 1) // b * b


def _gather_kernel(idx_ref, w_hbm, out_ref, sems, *, tb, chunk, slots):
    base = pl.program_id(0) * tb
    n_chunks = tb // chunk

    def issue(c):
        slot = c % slots
        for k in range(chunk):
            r = c * chunk + k
            row = idx_ref[base + r]
            pltpu.make_async_copy(
                w_hbm.at[pl.ds(row, 1), :],
                out_ref.at[pl.ds(r, 1), :],
                sems.at[slot],
            ).start(priority=c % 2)

    def wait(c):
        pltpu.make_async_copy(
            w_hbm.at[pl.ds(0, chunk), :],
            out_ref.at[pl.ds(c * chunk, chunk), :],
            sems.at[c % slots],
        ).wait()

    depth = min(slots - 1, n_chunks)
    for c in range(depth):
        issue(c)
    for c in range(n_chunks):
        if c + depth < n_chunks:
            issue(c + depth)
        wait(c)


def _lookup_hbm_gather(flat_ids, weights, tb):
    n_tok = flat_ids.shape[0]
    if n_tok > _MAX_TOKENS_PER_CALL:
        parts = [
            _lookup_hbm_gather(flat_ids[s:s + _MAX_TOKENS_PER_CALL], weights, tb)
            for s in range(0, n_tok, _MAX_TOKENS_PER_CALL)
        ]
        return jnp.concatenate(parts, axis=0)

    vsz, dsz = weights.shape
    n_pad = _round_up(n_tok, tb)
    nb = n_pad // tb
    if tb % _CHUNK == 0:
        chunk = _CHUNK
    elif tb % 32 == 0:
        chunk = 32
    else:
        chunk = _SUBLANE

    ids = jnp.pad(flat_ids, (0, n_pad - n_tok), constant_values=_PAD_IDX)

    out = pl.pallas_call(
        functools.partial(_gather_kernel, tb=tb, chunk=chunk, slots=_SLOTS),
        out_shape=jax.ShapeDtypeStruct((n_pad, dsz), weights.dtype),
        grid_spec=pltpu.PrefetchScalarGridSpec(
            num_scalar_prefetch=1,
            grid=(nb,),
            in_specs=[pl.BlockSpec(memory_space=pl.ANY)],
            out_specs=pl.BlockSpec((tb, dsz), lambda i, idx: (i, 0)),
            scratch_shapes=[pltpu.SemaphoreType.DMA((_SLOTS,))],
        ),
        compiler_params=pltpu.CompilerParams(
            dimension_semantics=("parallel",),
        ),
    )(ids, weights)
    return out[:n_tok]


def kernel(x, weights):
    B, T = x.shape
    vsz, dsz = weights.shape

    flat_ids = jnp.clip(x.reshape(-1).astype(jnp.int32), 0, vsz - 1)
    n_tok = flat_ids.shape[0]

    tb = _round_up(min(_TB, _round_up(n_tok, _SUBLANE)), _SUBLANE)
    out_flat = _lookup_hbm_gather(flat_ids, weights, tb)
    return out_flat.reshape(B, T, dsz)


# speedup vs baseline: 2.7081x; 1.0872x over previous
import functools

import jax
import jax.numpy as jnp
from jax.experimental import pallas as pl
from jax.experimental.pallas import tpu as pltpu

_PAD_IDX = 0
_SUBLANE = 8
_TB = 2048
_CHUNK = 16
_SLOTS = 16
_MAX_TOKENS_PER_CALL = 32768


def _round_up(a, b):
    return (a + b - 1) // b * b


def _gather_kernel(idx_ref, w_hbm, out_ref, sems, *, tb, chunk, slots):
    base = pl.program_id(0) * tb
    n_chunks = tb // chunk

    def issue(c):
        slot = c % slots
        for k in range(chunk):
            r = c * chunk + k
            row = idx_ref[base + r]
            pltpu.make_async_copy(
                w_hbm.at[pl.ds(row, 1), :],
                out_ref.at[pl.ds(r, 1), :],
                sems.at[slot],
            ).start(priority=c % 2)

    def wait(c):
        pltpu.make_async_copy(
            w_hbm.at[pl.ds(0, chunk), :],
            out_ref.at[pl.ds(c * chunk, chunk), :],
            sems.at[c % slots],
        ).wait()

    depth = min(slots - 1, n_chunks)
    for c in range(depth):
        issue(c)
    for c in range(n_chunks):
        if c + depth < n_chunks:
            issue(c + depth)
        wait(c)


def _lookup_hbm_gather(flat_ids, weights, tb):
    n_tok = flat_ids.shape[0]
    if n_tok > _MAX_TOKENS_PER_CALL:
        parts = [
            _lookup_hbm_gather(flat_ids[s:s + _MAX_TOKENS_PER_CALL], weights, tb)
            for s in range(0, n_tok, _MAX_TOKENS_PER_CALL)
        ]
        return jnp.concatenate(parts, axis=0)

    vsz, dsz = weights.shape
    n_pad = _round_up(n_tok, tb)
    nb = n_pad // tb
    if tb % _CHUNK == 0:
        chunk = _CHUNK
    elif tb % 32 == 0:
        chunk = 32
    else:
        chunk = _SUBLANE

    ids = jnp.pad(flat_ids, (0, n_pad - n_tok), constant_values=_PAD_IDX)

    out = pl.pallas_call(
        functools.partial(_gather_kernel, tb=tb, chunk=chunk, slots=_SLOTS),
        out_shape=jax.ShapeDtypeStruct((n_pad, dsz), weights.dtype),
        grid_spec=pltpu.PrefetchScalarGridSpec(
            num_scalar_prefetch=1,
            grid=(nb,),
            in_specs=[pl.BlockSpec(memory_space=pl.ANY)],
            out_specs=pl.BlockSpec((tb, dsz), lambda i, idx: (i, 0)),
            scratch_shapes=[pltpu.SemaphoreType.DMA((_SLOTS,))],
        ),
        compiler_params=pltpu.CompilerParams(
            dimension_semantics=("parallel",),
        ),
    )(ids, weights)
    return out[:n_tok]


def kernel(x, weights):
    B, T = x.shape
    vsz, dsz = weights.shape

    flat_ids = jnp.clip(x.reshape(-1).astype(jnp.int32), 0, vsz - 1)
    n_tok = flat_ids.shape[0]

    tb = _round_up(min(_TB, _round_up(n_tok, _SUBLANE)), _SUBLANE)
    out_flat = _lookup_hbm_gather(flat_ids, weights, tb)
    return out_flat.reshape(B, T, dsz)


# speedup vs baseline: 2.8358x; 1.0472x over previous
import functools

import jax
import jax.numpy as jnp
from jax.experimental import pallas as pl
from jax.experimental.pallas import tpu as pltpu

_PAD_IDX = 0
_SUBLANE = 8
_TB = 4096
_CHUNK = 16
_SLOTS = 16
_MAX_TOKENS_PER_CALL = 32768


def _round_up(a, b):
    return (a + b - 1) // b * b


def _gather_kernel(idx_ref, w_hbm, out_ref, sems, *, tb, chunk, slots):
    base = pl.program_id(0) * tb
    n_chunks = tb // chunk

    def issue(c):
        slot = c % slots
        for k in range(chunk):
            r = c * chunk + k
            row = idx_ref[base + r]
            pltpu.make_async_copy(
                w_hbm.at[pl.ds(row, 1), :],
                out_ref.at[pl.ds(r, 1), :],
                sems.at[slot],
            ).start(priority=c % 2)

    def wait(c):
        pltpu.make_async_copy(
            w_hbm.at[pl.ds(0, chunk), :],
            out_ref.at[pl.ds(c * chunk, chunk), :],
            sems.at[c % slots],
        ).wait()

    depth = min(slots - 1, n_chunks)
    for c in range(depth):
        issue(c)
    for c in range(n_chunks):
        if c + depth < n_chunks:
            issue(c + depth)
        wait(c)


def _lookup_hbm_gather(flat_ids, weights, tb):
    n_tok = flat_ids.shape[0]
    if n_tok > _MAX_TOKENS_PER_CALL:
        parts = [
            _lookup_hbm_gather(flat_ids[s:s + _MAX_TOKENS_PER_CALL], weights, tb)
            for s in range(0, n_tok, _MAX_TOKENS_PER_CALL)
        ]
        return jnp.concatenate(parts, axis=0)

    vsz, dsz = weights.shape
    n_pad = _round_up(n_tok, tb)
    nb = n_pad // tb
    if tb % _CHUNK == 0:
        chunk = _CHUNK
    elif tb % 32 == 0:
        chunk = 32
    else:
        chunk = _SUBLANE

    ids = jnp.pad(flat_ids, (0, n_pad - n_tok), constant_values=_PAD_IDX)

    out = pl.pallas_call(
        functools.partial(_gather_kernel, tb=tb, chunk=chunk, slots=_SLOTS),
        out_shape=jax.ShapeDtypeStruct((n_pad, dsz), weights.dtype),
        grid_spec=pltpu.PrefetchScalarGridSpec(
            num_scalar_prefetch=1,
            grid=(nb,),
            in_specs=[pl.BlockSpec(memory_space=pl.ANY)],
            out_specs=pl.BlockSpec((tb, dsz), lambda i, idx: (i, 0)),
            scratch_shapes=[pltpu.SemaphoreType.DMA((_SLOTS,))],
        ),
        compiler_params=pltpu.CompilerParams(
            dimension_semantics=("parallel",),
        ),
    )(ids, weights)
    return out[:n_tok]


def kernel(x, weights):
    B, T = x.shape
    vsz, dsz = weights.shape

    flat_ids = jnp.clip(x.reshape(-1).astype(jnp.int32), 0, vsz - 1)
    n_tok = flat_ids.shape[0]

    tb = _round_up(min(_TB, _round_up(n_tok, _SUBLANE)), _SUBLANE)
    out_flat = _lookup_hbm_gather(flat_ids, weights, tb)
    return out_flat.reshape(B, T, dsz)


# speedup vs baseline: 3.3842x; 1.1934x over previous
import functools

import jax
import jax.numpy as jnp
from jax.experimental import pallas as pl
from jax.experimental.pallas import tpu as pltpu

_PAD_IDX = 0
_SUBLANE = 8
_TB = 4096
_CHUNK = 16
_SLOTS = 32
_MAX_TOKENS_PER_CALL = 32768


def _round_up(a, b):
    return (a + b - 1) // b * b


def _gather_kernel(idx_ref, w_hbm, out_ref, sems, *, tb, chunk, slots):
    base = pl.program_id(0) * tb
    n_chunks = tb // chunk

    def issue(c):
        slot = c % slots
        for k in range(chunk):
            r = c * chunk + k
            row = idx_ref[base + r]
            pltpu.make_async_copy(
                w_hbm.at[pl.ds(row, 1), :],
                out_ref.at[pl.ds(r, 1), :],
                sems.at[slot],
            ).start(priority=c % 2)

    def wait(c):
        pltpu.make_async_copy(
            w_hbm.at[pl.ds(0, chunk), :],
            out_ref.at[pl.ds(c * chunk, chunk), :],
            sems.at[c % slots],
        ).wait()

    depth = min(slots - 1, n_chunks)
    for c in range(depth):
        issue(c)
    for c in range(n_chunks):
        if c + depth < n_chunks:
            issue(c + depth)
        wait(c)


def _lookup_hbm_gather(flat_ids, weights, tb):
    n_tok = flat_ids.shape[0]
    if n_tok > _MAX_TOKENS_PER_CALL:
        parts = [
            _lookup_hbm_gather(flat_ids[s:s + _MAX_TOKENS_PER_CALL], weights, tb)
            for s in range(0, n_tok, _MAX_TOKENS_PER_CALL)
        ]
        return jnp.concatenate(parts, axis=0)

    vsz, dsz = weights.shape
    n_pad = _round_up(n_tok, tb)
    nb = n_pad // tb
    if tb % _CHUNK == 0:
        chunk = _CHUNK
    elif tb % 32 == 0:
        chunk = 32
    else:
        chunk = _SUBLANE

    ids = jnp.pad(flat_ids, (0, n_pad - n_tok), constant_values=_PAD_IDX)

    out = pl.pallas_call(
        functools.partial(_gather_kernel, tb=tb, chunk=chunk, slots=_SLOTS),
        out_shape=jax.ShapeDtypeStruct((n_pad, dsz), weights.dtype),
        grid_spec=pltpu.PrefetchScalarGridSpec(
            num_scalar_prefetch=1,
            grid=(nb,),
            in_specs=[pl.BlockSpec(memory_space=pl.ANY)],
            out_specs=pl.BlockSpec((tb, dsz), lambda i, idx: (i, 0)),
            scratch_shapes=[pltpu.SemaphoreType.DMA((_SLOTS,))],
        ),
        compiler_params=pltpu.CompilerParams(
            dimension_semantics=("parallel",),
        ),
    )(ids, weights)
    return out[:n_tok]


def kernel(x, weights):
    B, T = x.shape
    vsz, dsz = weights.shape

    flat_ids = jnp.clip(x.reshape(-1).astype(jnp.int32), 0, vsz - 1)
    n_tok = flat_ids.shape[0]

    tb = _round_up(min(_TB, _round_up(n_tok, _SUBLANE)), _SUBLANE)
    out_flat = _lookup_hbm_gather(flat_ids, weights, tb)
    return out_flat.reshape(B, T, dsz)


# speedup vs baseline: 3.5681x; 1.0543x over previous
import functools

import jax
import jax.numpy as jnp
from jax.experimental import pallas as pl
from jax.experimental.pallas import tpu as pltpu

_PAD_IDX = 0
_SUBLANE = 8
_TB = 4096
_CHUNK = 16
_SLOTS = 64
_MAX_TOKENS_PER_CALL = 32768


def _round_up(a, b):
    return (a + b - 1) // b * b


def _gather_kernel(idx_ref, w_hbm, out_ref, sems, *, tb, chunk, slots):
    base = pl.program_id(0) * tb
    n_chunks = tb // chunk

    def issue(c):
        slot = c % slots
        for k in range(chunk):
            r = c * chunk + k
            row = idx_ref[base + r]
            pltpu.make_async_copy(
                w_hbm.at[pl.ds(row, 1), :],
                out_ref.at[pl.ds(r, 1), :],
                sems.at[slot],
            ).start(priority=c % 2)

    def wait(c):
        pltpu.make_async_copy(
            w_hbm.at[pl.ds(0, chunk), :],
            out_ref.at[pl.ds(c * chunk, chunk), :],
            sems.at[c % slots],
        ).wait()

    depth = min(slots - 1, n_chunks)
    for c in range(depth):
        issue(c)
    for c in range(n_chunks):
        if c + depth < n_chunks:
            issue(c + depth)
        wait(c)


def _lookup_hbm_gather(flat_ids, weights, tb):
    n_tok = flat_ids.shape[0]
    if n_tok > _MAX_TOKENS_PER_CALL:
        parts = [
            _lookup_hbm_gather(flat_ids[s:s + _MAX_TOKENS_PER_CALL], weights, tb)
            for s in range(0, n_tok, _MAX_TOKENS_PER_CALL)
        ]
        return jnp.concatenate(parts, axis=0)

    vsz, dsz = weights.shape
    n_pad = _round_up(n_tok, tb)
    nb = n_pad // tb
    if tb % _CHUNK == 0:
        chunk = _CHUNK
    elif tb % 32 == 0:
        chunk = 32
    else:
        chunk = _SUBLANE

    ids = jnp.pad(flat_ids, (0, n_pad - n_tok), constant_values=_PAD_IDX)

    out = pl.pallas_call(
        functools.partial(_gather_kernel, tb=tb, chunk=chunk, slots=_SLOTS),
        out_shape=jax.ShapeDtypeStruct((n_pad, dsz), weights.dtype),
        grid_spec=pltpu.PrefetchScalarGridSpec(
            num_scalar_prefetch=1,
            grid=(nb,),
            in_specs=[pl.BlockSpec(memory_space=pl.ANY)],
            out_specs=pl.BlockSpec((tb, dsz), lambda i, idx: (i, 0)),
            scratch_shapes=[pltpu.SemaphoreType.DMA((_SLOTS,))],
        ),
        compiler_params=pltpu.CompilerParams(
            dimension_semantics=("parallel",),
        ),
    )(ids, weights)
    return out[:n_tok]


def kernel(x, weights):
    B, T = x.shape
    vsz, dsz = weights.shape

    flat_ids = jnp.clip(x.reshape(-1).astype(jnp.int32), 0, vsz - 1)
    n_tok = flat_ids.shape[0]

    tb = _round_up(min(_TB, _round_up(n_tok, _SUBLANE)), _SUBLANE)
    out_flat = _lookup_hbm_gather(flat_ids, weights, tb)
    return out_flat.reshape(B, T, dsz)


# speedup vs baseline: 3.5853x; 1.0048x over previous
import functools

import jax
import jax.numpy as jnp
from jax.experimental import pallas as pl
from jax.experimental.pallas import tpu as pltpu

_PAD_IDX = 0
_SUBLANE = 8
_TB = 4096
_CHUNK = 16
_SLOTS = 128
_MAX_TOKENS_PER_CALL = 32768


def _round_up(a, b):
    return (a + b - 1) // b * b


def _gather_kernel(idx_ref, w_hbm, out_ref, sems, *, tb, chunk, slots):
    base = pl.program_id(0) * tb
    n_chunks = tb // chunk

    def issue(c):
        slot = c % slots
        for k in range(chunk):
            r = c * chunk + k
            row = idx_ref[base + r]
            pltpu.make_async_copy(
                w_hbm.at[pl.ds(row, 1), :],
                out_ref.at[pl.ds(r, 1), :],
                sems.at[slot],
            ).start(priority=c % 2)

    def wait(c):
        pltpu.make_async_copy(
            w_hbm.at[pl.ds(0, chunk), :],
            out_ref.at[pl.ds(c * chunk, chunk), :],
            sems.at[c % slots],
        ).wait()

    depth = min(slots - 1, n_chunks)
    for c in range(depth):
        issue(c)
    for c in range(n_chunks):
        if c + depth < n_chunks:
            issue(c + depth)
        wait(c)


def _lookup_hbm_gather(flat_ids, weights, tb):
    n_tok = flat_ids.shape[0]
    if n_tok > _MAX_TOKENS_PER_CALL:
        parts = [
            _lookup_hbm_gather(flat_ids[s:s + _MAX_TOKENS_PER_CALL], weights, tb)
            for s in range(0, n_tok, _MAX_TOKENS_PER_CALL)
        ]
        return jnp.concatenate(parts, axis=0)

    vsz, dsz = weights.shape
    n_pad = _round_up(n_tok, tb)
    nb = n_pad // tb
    if tb % _CHUNK == 0:
        chunk = _CHUNK
    elif tb % 32 == 0:
        chunk = 32
    else:
        chunk = _SUBLANE

    ids = jnp.pad(flat_ids, (0, n_pad - n_tok), constant_values=_PAD_IDX)

    out = pl.pallas_call(
        functools.partial(_gather_kernel, tb=tb, chunk=chunk, slots=_SLOTS),
        out_shape=jax.ShapeDtypeStruct((n_pad, dsz), weights.dtype),
        grid_spec=pltpu.PrefetchScalarGridSpec(
            num_scalar_prefetch=1,
            grid=(nb,),
            in_specs=[pl.BlockSpec(memory_space=pl.ANY)],
            out_specs=pl.BlockSpec((tb, dsz), lambda i, idx: (i, 0)),
            scratch_shapes=[pltpu.SemaphoreType.DMA((_SLOTS,))],
        ),
        compiler_params=pltpu.CompilerParams(
            dimension_semantics=("parallel",),
        ),
    )(ids, weights)
    return out[:n_tok]


def kernel(x, weights):
    B, T = x.shape
    vsz, dsz = weights.shape

    flat_ids = jnp.clip(x.reshape(-1).astype(jnp.int32), 0, vsz - 1)
    n_tok = flat_ids.shape[0]

    tb = _round_up(min(_TB, _round_up(n_tok, _SUBLANE)), _SUBLANE)
    out_flat = _lookup_hbm_gather(flat_ids, weights, tb)
    return out_flat.reshape(B, T, dsz)


# speedup vs baseline: 3.6300x; 1.0125x over previous
import functools

import jax
import jax.numpy as jnp
from jax.experimental import pallas as pl
from jax.experimental.pallas import tpu as pltpu

_PAD_IDX = 0
_SUBLANE = 8
_TB = 4096
_CHUNK = 32
_SLOTS = 64
_MAX_TOKENS_PER_CALL = 32768


def _round_up(a, b):
    return (a + b - 1) // b * b


def _gather_kernel(idx_ref, w_hbm, out_ref, sems, *, tb, chunk, slots):
    base = pl.program_id(0) * tb
    n_chunks = tb // chunk

    def issue(c):
        slot = c % slots
        for k in range(chunk):
            r = c * chunk + k
            row = idx_ref[base + r]
            pltpu.make_async_copy(
                w_hbm.at[pl.ds(row, 1), :],
                out_ref.at[pl.ds(r, 1), :],
                sems.at[slot],
            ).start(priority=c % 2)

    def wait(c):
        pltpu.make_async_copy(
            w_hbm.at[pl.ds(0, chunk), :],
            out_ref.at[pl.ds(c * chunk, chunk), :],
            sems.at[c % slots],
        ).wait()

    depth = min(slots - 1, n_chunks)
    for c in range(depth):
        issue(c)
    for c in range(n_chunks):
        if c + depth < n_chunks:
            issue(c + depth)
        wait(c)


def _lookup_hbm_gather(flat_ids, weights, tb):
    n_tok = flat_ids.shape[0]
    if n_tok > _MAX_TOKENS_PER_CALL:
        parts = [
            _lookup_hbm_gather(flat_ids[s:s + _MAX_TOKENS_PER_CALL], weights, tb)
            for s in range(0, n_tok, _MAX_TOKENS_PER_CALL)
        ]
        return jnp.concatenate(parts, axis=0)

    vsz, dsz = weights.shape
    n_pad = _round_up(n_tok, tb)
    nb = n_pad // tb
    if tb % _CHUNK == 0:
        chunk = _CHUNK
    elif tb % 32 == 0:
        chunk = 32
    else:
        chunk = _SUBLANE

    ids = jnp.pad(flat_ids, (0, n_pad - n_tok), constant_values=_PAD_IDX)

    out = pl.pallas_call(
        functools.partial(_gather_kernel, tb=tb, chunk=chunk, slots=_SLOTS),
        out_shape=jax.ShapeDtypeStruct((n_pad, dsz), weights.dtype),
        grid_spec=pltpu.PrefetchScalarGridSpec(
            num_scalar_prefetch=1,
            grid=(nb,),
            in_specs=[pl.BlockSpec(memory_space=pl.ANY)],
            out_specs=pl.BlockSpec((tb, dsz), lambda i, idx: (i, 0)),
            scratch_shapes=[pltpu.SemaphoreType.DMA((_SLOTS,))],
        ),
        compiler_params=pltpu.CompilerParams(
            dimension_semantics=("parallel",),
        ),
    )(ids, weights)
    return out[:n_tok]


def kernel(x, weights):
    B, T = x.shape
    vsz, dsz = weights.shape

    flat_ids = jnp.clip(x.reshape(-1).astype(jnp.int32), 0, vsz - 1)
    n_tok = flat_ids.shape[0]

    tb = _round_up(min(_TB, _round_up(n_tok, _SUBLANE)), _SUBLANE)
    out_flat = _lookup_hbm_gather(flat_ids, weights, tb)
    return out_flat.reshape(B, T, dsz)


# speedup vs baseline: 3.6823x; 1.0144x over previous
import functools

import jax
import jax.numpy as jnp
from jax.experimental import pallas as pl
from jax.experimental.pallas import tpu as pltpu

_PAD_IDX = 0
_SUBLANE = 8
_TB = 4096
_CHUNK = 32
_SLOTS = 128
_MAX_TOKENS_PER_CALL = 32768


def _round_up(a, b):
    return (a + b - 1) // b * b


def _gather_kernel(idx_ref, w_hbm, out_ref, sems, *, tb, chunk, slots):
    base = pl.program_id(0) * tb
    n_chunks = tb // chunk

    def issue(c):
        slot = c % slots
        for k in range(chunk):
            r = c * chunk + k
            row = idx_ref[base + r]
            pltpu.make_async_copy(
                w_hbm.at[pl.ds(row, 1), :],
                out_ref.at[pl.ds(r, 1), :],
                sems.at[slot],
            ).start(priority=c % 2)

    def wait(c):
        pltpu.make_async_copy(
            w_hbm.at[pl.ds(0, chunk), :],
            out_ref.at[pl.ds(c * chunk, chunk), :],
            sems.at[c % slots],
        ).wait()

    depth = min(slots - 1, n_chunks)
    for c in range(depth):
        issue(c)
    for c in range(n_chunks):
        if c + depth < n_chunks:
            issue(c + depth)
        wait(c)


def _lookup_hbm_gather(flat_ids, weights, tb):
    n_tok = flat_ids.shape[0]
    if n_tok > _MAX_TOKENS_PER_CALL:
        parts = [
            _lookup_hbm_gather(flat_ids[s:s + _MAX_TOKENS_PER_CALL], weights, tb)
            for s in range(0, n_tok, _MAX_TOKENS_PER_CALL)
        ]
        return jnp.concatenate(parts, axis=0)

    vsz, dsz = weights.shape
    n_pad = _round_up(n_tok, tb)
    nb = n_pad // tb
    if tb % _CHUNK == 0:
        chunk = _CHUNK
    elif tb % 32 == 0:
        chunk = 32
    else:
        chunk = _SUBLANE

    ids = jnp.pad(flat_ids, (0, n_pad - n_tok), constant_values=_PAD_IDX)

    out = pl.pallas_call(
        functools.partial(_gather_kernel, tb=tb, chunk=chunk, slots=_SLOTS),
        out_shape=jax.ShapeDtypeStruct((n_pad, dsz), weights.dtype),
        grid_spec=pltpu.PrefetchScalarGridSpec(
            num_scalar_prefetch=1,
            grid=(nb,),
            in_specs=[pl.BlockSpec(memory_space=pl.ANY)],
            out_specs=pl.BlockSpec((tb, dsz), lambda i, idx: (i, 0)),
            scratch_shapes=[pltpu.SemaphoreType.DMA((_SLOTS,))],
        ),
        compiler_params=pltpu.CompilerParams(
            dimension_semantics=("parallel",),
        ),
    )(ids, weights)
    return out[:n_tok]


def kernel(x, weights):
    B, T = x.shape
    vsz, dsz = weights.shape

    flat_ids = jnp.clip(x.reshape(-1).astype(jnp.int32), 0, vsz - 1)
    n_tok = flat_ids.shape[0]

    tb = _round_up(min(_TB, _round_up(n_tok, _SUBLANE)), _SUBLANE)
    out_flat = _lookup_hbm_gather(flat_ids, weights, tb)
    return out_flat.reshape(B, T, dsz)
